# baseline (speedup 1.0000x reference)
import functools

import jax
import jax.numpy as jnp
from jax import lax
from jax.experimental import pallas as pl
from jax.experimental.pallas import tpu as pltpu
from jax.experimental.pallas import tpu_sc as plsc

_NUM_SLICES = 16
_SLICE_W = 256

_PERM = (7, 4, 2, 5, 3, 6, 10, 11, 15, 8, 9, 13, 14, 0, 1, 12)


def _runs(perm):
    runs, j = [], 0
    while j < len(perm):
        k = j + 1
        while k < len(perm) and perm[k] == perm[k - 1] + 1:
            k += 1
        runs.append((j, perm[j], k - j))
        j = k
    return tuple(runs)


_ALL_RUNS = _runs(_PERM)
_HALF_SL = _NUM_SLICES // 2
_HRUNS = (
    tuple(r for r in _ALL_RUNS if r[0] < _HALF_SL),
    tuple(r for r in _ALL_RUNS if r[0] >= _HALF_SL),
)
assert all(d + ln <= _HALF_SL or d >= _HALF_SL for d, _, ln in _ALL_RUNS)

_NC, _NS = 2, 16
_NW = _NC * _NS
_ROWS = 2 * 4096
_COLS = 4096
_HCOLS = _COLS // 2
_RPW = _ROWS // _NW

_R = 8
_CHUNKS = _RPW // _R

_MESH = plsc.VectorSubcoreMesh(
    core_axis_name="c", subcore_axis_name="s", num_cores=_NC, num_subcores=_NS
)


@functools.partial(
    pl.kernel,
    out_type=jax.ShapeDtypeStruct((_ROWS, _COLS), jnp.float32),
    mesh=_MESH,
    scratch_types=[
        pltpu.VMEM((2, 2, _R, _HCOLS), jnp.float32),
        pltpu.SemaphoreType.DMA((2, 2)),
        pltpu.SemaphoreType.DMA((2, 2)),
    ],
    compiler_params=pltpu.CompilerParams(use_tc_tiling_on_sc=True),
)
def _shuffle(in_hbm, out_hbm, bufs, si, so):
    wid = lax.axis_index("s") * _NC + lax.axis_index("c")
    base = wid * _RPW

    def start_in(c, b, h):
        row = base + c * _R
        for dst, src, ln in _HRUNS[h]:
            pltpu.make_async_copy(
                in_hbm.at[pl.ds(row, _R), pl.ds(src * _SLICE_W, ln * _SLICE_W)],
                bufs.at[h, b, :, pl.ds((dst - h * _HALF_SL) * _SLICE_W, ln * _SLICE_W)],
                si.at[h, b],
            ).start()

    def wait_in(c, b, h):
        row = base + c * _R
        pltpu.make_async_copy(
            in_hbm.at[pl.ds(row, _R), pl.ds(h * _HCOLS, _HCOLS)],
            bufs.at[h, b],
            si.at[h, b],
        ).wait()

    def out_copy(c, b, h):
        row = base + c * _R
        return pltpu.make_async_copy(
            bufs.at[h, b],
            out_hbm.at[pl.ds(row, _R), pl.ds(h * _HCOLS, _HCOLS)],
            so.at[h, b],
        )

    for h in range(2):
        start_in(0, 0, h)
        start_in(1, 1, h)
    for h in range(2):
        wait_in(0, 0, h)
        out_copy(0, 0, h).start()

    @pl.loop(1, _CHUNKS - 1, step=2)
    def _pipe(p):
        for off in range(2):
            c = p + off
            b = (1 + off) % 2
            bp = (0 + off) % 2
            for h in range(2):
                out_copy(c - 1, bp, h).wait()
                start_in(c + 1, bp, h)
                wait_in(c, b, h)
                out_copy(c, b, h).start()

    c = _CHUNKS - 1
    for h in range(2):
        out_copy(c - 1, 0, h).wait()
        wait_in(c, 1, h)
        out_copy(c, 1, h).start()
    for h in range(2):
        out_copy(c, 1, h).wait()


def kernel(x):
    shape = x.shape
    x2 = x.reshape(_ROWS, _COLS)
    out = _shuffle(x2)
    return out.reshape(shape)


# speedup vs baseline: 1.0056x; 1.0056x over previous
import functools

import jax
import jax.numpy as jnp
from jax import lax
from jax.experimental import pallas as pl
from jax.experimental.pallas import tpu as pltpu
from jax.experimental.pallas import tpu_sc as plsc

_NUM_SLICES = 16
_SLICE_W = 256

_PERM = (7, 4, 2, 5, 3, 6, 10, 11, 15, 8, 9, 13, 14, 0, 1, 12)


def _runs(perm):
    runs, j = [], 0
    while j < len(perm):
        k = j + 1
        while k < len(perm) and perm[k] == perm[k - 1] + 1:
            k += 1
        runs.append((j, perm[j], k - j))
        j = k
    return tuple(runs)


_RUNS = _runs(_PERM)

_NC, _NS = 2, 16
_NW = _NC * _NS
_ROWS = 2 * 4096
_COLS = 4096
_RPW = _ROWS // _NW

_R = 8
_CHUNKS = _RPW // _R
_NBUF = 3

_MESH = plsc.VectorSubcoreMesh(
    core_axis_name="c", subcore_axis_name="s", num_cores=_NC, num_subcores=_NS
)


@functools.partial(
    pl.kernel,
    out_type=jax.ShapeDtypeStruct((_ROWS, _COLS), jnp.float32),
    mesh=_MESH,
    scratch_types=[
        pltpu.VMEM((_R, _COLS), jnp.float32),
        pltpu.VMEM((_R, _COLS), jnp.float32),
        pltpu.VMEM((_R, _COLS), jnp.float32),
        pltpu.SemaphoreType.DMA,
        pltpu.SemaphoreType.DMA,
        pltpu.SemaphoreType.DMA,
        pltpu.SemaphoreType.DMA,
        pltpu.SemaphoreType.DMA,
        pltpu.SemaphoreType.DMA,
    ],
    compiler_params=pltpu.CompilerParams(use_tc_tiling_on_sc=True),
)
def _shuffle(in_hbm, out_hbm, buf0, buf1, buf2, si0, si1, si2, so0, so1, so2):
    wid = lax.axis_index("s") * _NC + lax.axis_index("c")
    base = wid * _RPW
    bufs = (buf0, buf1, buf2)
    sis = (si0, si1, si2)
    sos = (so0, so1, so2)

    def start_in(c, b):
        row = base + c * _R
        for dst, src, ln in _RUNS:
            pltpu.make_async_copy(
                in_hbm.at[pl.ds(row, _R), pl.ds(src * _SLICE_W, ln * _SLICE_W)],
                bufs[b].at[:, pl.ds(dst * _SLICE_W, ln * _SLICE_W)],
                sis[b],
            ).start()

    def wait_in(c, b):
        row = base + c * _R
        pltpu.make_async_copy(in_hbm.at[pl.ds(row, _R)], bufs[b], sis[b]).wait()

    def out_copy(c, b):
        row = base + c * _R
        return pltpu.make_async_copy(bufs[b], out_hbm.at[pl.ds(row, _R)], sos[b])

    for b in range(_NBUF):
        start_in(b, b)
    wait_in(0, 0)
    out_copy(0, 0).start()

    @pl.loop(1, _CHUNKS - 4, step=_NBUF)
    def _pipe(p):
        for off in range(_NBUF):
            c = p + off
            b = (1 + off) % _NBUF
            bp = (0 + off) % _NBUF
            out_copy(c - 1, bp).wait()
            start_in(c + 2, bp)
            wait_in(c, b)
            out_copy(c, b).start()

    for c in range(_CHUNKS - 4, _CHUNKS):
        b = c % _NBUF
        bp = (c - 1) % _NBUF
        out_copy(c - 1, bp).wait()
        if c + 2 < _CHUNKS:
            start_in(c + 2, bp)
        wait_in(c, b)
        out_copy(c, b).start()
    out_copy(_CHUNKS - 1, (_CHUNKS - 1) % _NBUF).wait()


def kernel(x):
    shape = x.shape
    x2 = x.reshape(_ROWS, _COLS)
    out = _shuffle(x2)
    return out.reshape(shape)


# speedup vs baseline: 1.0094x; 1.0038x over previous
import functools

import jax
import jax.numpy as jnp
from jax import lax
from jax.experimental import pallas as pl
from jax.experimental.pallas import tpu as pltpu
from jax.experimental.pallas import tpu_sc as plsc

_NUM_SLICES = 16
_SLICE_W = 256

_PERM = (7, 4, 2, 5, 3, 6, 10, 11, 15, 8, 9, 13, 14, 0, 1, 12)


def _runs(perm):
    runs, j = [], 0
    while j < len(perm):
        k = j + 1
        while k < len(perm) and perm[k] == perm[k - 1] + 1:
            k += 1
        runs.append((j, perm[j], k - j))
        j = k
    return tuple(runs)


_RUNS = _runs(_PERM)

_NC, _NS = 2, 16
_NW = _NC * _NS
_ROWS = 2 * 4096
_COLS = 4096
_RPW = _ROWS // _NW

_R = 8
_CHUNKS = _RPW // _R
_NBUF = 3

_MESH = plsc.VectorSubcoreMesh(
    core_axis_name="c", subcore_axis_name="s", num_cores=_NC, num_subcores=_NS
)


@functools.partial(
    pl.kernel,
    out_type=jax.ShapeDtypeStruct((_ROWS, _COLS), jnp.float32),
    mesh=_MESH,
    scratch_types=[
        pltpu.VMEM((_R, _COLS), jnp.float32),
        pltpu.VMEM((_R, _COLS), jnp.float32),
        pltpu.VMEM((_R, _COLS), jnp.float32),
        pltpu.SemaphoreType.DMA,
        pltpu.SemaphoreType.DMA,
        pltpu.SemaphoreType.DMA,
        pltpu.SemaphoreType.DMA,
        pltpu.SemaphoreType.DMA,
        pltpu.SemaphoreType.DMA,
    ],
    compiler_params=pltpu.CompilerParams(use_tc_tiling_on_sc=True),
)
def _shuffle(in_hbm, out_hbm, buf0, buf1, buf2, si0, si1, si2, so0, so1, so2):
    wid = lax.axis_index("c") * _NS + lax.axis_index("s")
    base = wid * _RPW
    bufs = (buf0, buf1, buf2)
    sis = (si0, si1, si2)
    sos = (so0, so1, so2)

    def start_in(c, b):
        row = base + c * _R
        for dst, src, ln in _RUNS:
            pltpu.make_async_copy(
                in_hbm.at[pl.ds(row, _R), pl.ds(src * _SLICE_W, ln * _SLICE_W)],
                bufs[b].at[:, pl.ds(dst * _SLICE_W, ln * _SLICE_W)],
                sis[b],
            ).start()

    def wait_in(c, b):
        row = base + c * _R
        pltpu.make_async_copy(in_hbm.at[pl.ds(row, _R)], bufs[b], sis[b]).wait()

    def out_copy(c, b):
        row = base + c * _R
        return pltpu.make_async_copy(bufs[b], out_hbm.at[pl.ds(row, _R)], sos[b])

    for b in range(_NBUF):
        start_in(b, b)
    wait_in(0, 0)
    out_copy(0, 0).start()

    @pl.loop(1, _CHUNKS - 4, step=_NBUF)
    def _pipe(p):
        for off in range(_NBUF):
            c = p + off
            b = (1 + off) % _NBUF
            bp = (0 + off) % _NBUF
            out_copy(c - 1, bp).wait()
            start_in(c + 2, bp)
            wait_in(c, b)
            out_copy(c, b).start()

    for c in range(_CHUNKS - 4, _CHUNKS):
        b = c % _NBUF
        bp = (c - 1) % _NBUF
        out_copy(c - 1, bp).wait()
        if c + 2 < _CHUNKS:
            start_in(c + 2, bp)
        wait_in(c, b)
        out_copy(c, b).start()
    out_copy(_CHUNKS - 1, (_CHUNKS - 1) % _NBUF).wait()


def kernel(x):
    shape = x.shape
    x2 = x.reshape(_ROWS, _COLS)
    out = _shuffle(x2)
    return out.reshape(shape)
